# baseline (speedup 1.0000x reference)
import functools

import jax
import jax.numpy as jnp
from jax import lax
from jax.experimental import pallas as pl
from jax.experimental.pallas import tpu as pltpu
from jax.experimental.pallas import tpu_sc as plsc

L = 16
NBUF = 3
DEPTH = 2


@functools.partial(jax.jit, static_argnums=(3, 4))
def _lookup(ids_flat, original_weight, new_weight, n_orig, n_new):
    total = ids_flat.shape[0]
    d = original_weight.shape[1]

    info = plsc.get_sparse_core_info()
    nw = info.num_cores * info.num_subcores
    per_w = total // nw
    sub_rows = 128
    chunk = sub_rows
    n_chunk = per_w // chunk
    g_per_chunk = chunk // L

    mesh = plsc.VectorSubcoreMesh(core_axis_name="c", subcore_axis_name="s")

    @functools.partial(
        pl.kernel,
        mesh=mesh,
        compiler_params=pltpu.CompilerParams(needs_layout_passes=False),
        out_type=jax.ShapeDtypeStruct((total, d), jnp.float32),
        scratch_types=[
            pltpu.VMEM((n_new, d), jnp.float32),
            pltpu.VMEM((per_w,), jnp.int32),
            pltpu.VMEM((per_w,), jnp.int32),
            pltpu.SMEM((n_chunk,), jnp.int32),
            *([pltpu.VMEM((chunk, d), jnp.float32)] * NBUF),
            pltpu.SemaphoreType.DMA,
            pltpu.SemaphoreType.DMA,
            pltpu.SemaphoreType.DMA,
        ],
    )
    def k(ids_hbm, orig_hbm, new_hbm, out_hbm,
          new_v, raw_v, clip_v, flags_s, *rows_and_sems):
        rows = rows_and_sems[:NBUF]
        gsem, osem, nsem = rows_and_sems[NBUF:]
        wid = lax.axis_index("s") * info.num_cores + lax.axis_index("c")
        base = wid * per_w

        cp_new = pltpu.async_copy(new_hbm, new_v, nsem)
        pltpu.sync_copy(ids_hbm.at[pl.ds(base, per_w)], raw_v)

        lim = jnp.full((L,), n_orig - 1, jnp.int32)
        big = jnp.full((L,), n_orig, jnp.int32)
        zero = jnp.zeros((L,), jnp.int32)
        lane = jnp.arange(L, dtype=jnp.int32)
        one = jnp.full((L,), 1, jnp.int32)

        def clip_chunk(c):
            def clip_body(g, acc, c=c):
                off = c * chunk + g * L
                v = raw_v[pl.ds(off, L)]
                clip_v[pl.ds(off, L)] = jnp.minimum(v, lim)
                return acc + jnp.where(v >= big, one, zero)

            acc = lax.fori_loop(0, g_per_chunk, clip_body, zero)
            flags_s[c] = plsc.all_reduce_population_count(acc > 0)[0]

        def fire_gather(c):
            return [
                pltpu.async_copy(
                    orig_hbm.at[clip_v.at[pl.ds(c * chunk, chunk)]],
                    rows[c % NBUF], gsem),
            ]

        def fire_out(c):
            return pltpu.async_copy(
                rows[c % NBUF],
                out_hbm.at[pl.ds(base + c * chunk, chunk)], osem)

        def patch(c, buf):
            @pl.when(flags_s[c] > 0)
            def _():
                def group_body(g, carry):
                    v = raw_v[pl.ds(c * chunk + g * L, L)]
                    mask = v >= big

                    @pl.when(plsc.all_reduce_population_count(mask)[0] > 0)
                    def _():
                        nidx = jnp.minimum(
                            jnp.maximum(v - big, zero),
                            jnp.full((L,), n_new - 1, jnp.int32))
                        row_ids = jnp.full((L,), g * L, jnp.int32) + lane

                        def col_body(cb, cc):
                            cvec = jnp.full((L,), cb, jnp.int32)
                            vals = plsc.load_gather(new_v, [nidx, cvec],
                                                    mask=mask)
                            plsc.store_scatter(buf, [row_ids, cvec], vals,
                                               mask=mask)
                            return cc

                        lax.fori_loop(0, d, col_body, 0)

                    return carry

                lax.fori_loop(0, g_per_chunk, group_body, 0)

        gcp = [None] * n_chunk
        ocp = [None] * n_chunk
        head = min(DEPTH, n_chunk)
        for c in range(head):
            clip_chunk(c)
            gcp[c] = fire_gather(c)
        for c in range(head, n_chunk):
            clip_chunk(c)
        cp_new.wait()

        waited = set()
        for c in range(n_chunk):
            for cp in gcp[c]:
                cp.wait()
            patch(c, rows[c % NBUF])
            ocp[c] = fire_out(c)
            nxt = c + DEPTH
            if nxt < n_chunk:
                if nxt >= NBUF:
                    ocp[nxt - NBUF].wait()
                    waited.add(nxt - NBUF)
                gcp[nxt] = fire_gather(nxt)
        for j in range(n_chunk):
            if j not in waited:
                ocp[j].wait()

    return k(ids_flat, original_weight, new_weight)


def kernel(input_ids, original_weight, new_weight):
    b, s = input_ids.shape
    n_orig, d = original_weight.shape
    n_new = new_weight.shape[0]
    ids_flat = input_ids.reshape(-1).astype(jnp.int32)
    out = _lookup(ids_flat, original_weight, new_weight, n_orig, n_new)
    return out.reshape(b, s, d)


# speedup vs baseline: 1.0203x; 1.0203x over previous
import functools

import jax
import jax.numpy as jnp
from jax import lax
from jax.experimental import pallas as pl
from jax.experimental.pallas import tpu as pltpu
from jax.experimental.pallas import tpu_sc as plsc

L = 16
NBUF = 6
DEPTH = 6


@functools.partial(jax.jit, static_argnums=(3, 4))
def _lookup(ids_flat, original_weight, new_weight, n_orig, n_new):
    total = ids_flat.shape[0]
    d = original_weight.shape[1]

    info = plsc.get_sparse_core_info()
    nw = info.num_cores * info.num_subcores
    per_w = total // nw
    sub_rows = 128
    chunk = sub_rows
    n_chunk = per_w // chunk
    g_per_chunk = chunk // L

    mesh = plsc.VectorSubcoreMesh(core_axis_name="c", subcore_axis_name="s")

    @functools.partial(
        pl.kernel,
        mesh=mesh,
        compiler_params=pltpu.CompilerParams(needs_layout_passes=False),
        out_type=jax.ShapeDtypeStruct((total, d), jnp.float32),
        scratch_types=[
            pltpu.VMEM((n_new, d), jnp.float32),
            pltpu.VMEM((per_w,), jnp.int32),
            pltpu.VMEM((per_w,), jnp.int32),
            pltpu.SMEM((n_chunk,), jnp.int32),
            *([pltpu.VMEM((chunk, d), jnp.float32)] * NBUF),
            pltpu.SemaphoreType.DMA,
            pltpu.SemaphoreType.DMA,
            pltpu.SemaphoreType.DMA,
        ],
    )
    def k(ids_hbm, orig_hbm, new_hbm, out_hbm,
          new_v, raw_v, clip_v, flags_s, *rows_and_sems):
        rows = rows_and_sems[:NBUF]
        gsem, osem, nsem = rows_and_sems[NBUF:]
        wid = lax.axis_index("s") * info.num_cores + lax.axis_index("c")
        base = wid * per_w

        cp_new = pltpu.async_copy(new_hbm, new_v, nsem)
        pltpu.sync_copy(ids_hbm.at[pl.ds(base, per_w)], raw_v)

        lim = jnp.full((L,), n_orig - 1, jnp.int32)
        big = jnp.full((L,), n_orig, jnp.int32)
        zero = jnp.zeros((L,), jnp.int32)
        lane = jnp.arange(L, dtype=jnp.int32)
        one = jnp.full((L,), 1, jnp.int32)

        def clip_chunk(c):
            def clip_body(g, acc, c=c):
                off = c * chunk + g * L
                v = raw_v[pl.ds(off, L)]
                clip_v[pl.ds(off, L)] = jnp.minimum(v, lim)
                return acc + jnp.where(v >= big, one, zero)

            acc = lax.fori_loop(0, g_per_chunk, clip_body, zero)
            flags_s[c] = plsc.all_reduce_population_count(acc > 0)[0]

        def fire_gather(c):
            return [
                pltpu.async_copy(
                    orig_hbm.at[clip_v.at[pl.ds(c * chunk, chunk)]],
                    rows[c % NBUF], gsem),
            ]

        def fire_out(c):
            return pltpu.async_copy(
                rows[c % NBUF],
                out_hbm.at[pl.ds(base + c * chunk, chunk)], osem)

        def patch(c, buf):
            @pl.when(flags_s[c] > 0)
            def _():
                def group_body(g, carry):
                    v = raw_v[pl.ds(c * chunk + g * L, L)]
                    mask = v >= big

                    @pl.when(plsc.all_reduce_population_count(mask)[0] > 0)
                    def _():
                        nidx = jnp.minimum(
                            jnp.maximum(v - big, zero),
                            jnp.full((L,), n_new - 1, jnp.int32))
                        row_ids = jnp.full((L,), g * L, jnp.int32) + lane

                        def col_body(cb, cc):
                            cvec = jnp.full((L,), cb, jnp.int32)
                            vals = plsc.load_gather(new_v, [nidx, cvec],
                                                    mask=mask)
                            plsc.store_scatter(buf, [row_ids, cvec], vals,
                                               mask=mask)
                            return cc

                        lax.fori_loop(0, d, col_body, 0)

                    return carry

                lax.fori_loop(0, g_per_chunk, group_body, 0)

        gcp = [None] * n_chunk
        ocp = [None] * n_chunk
        head = min(DEPTH, n_chunk)
        for c in range(head):
            clip_chunk(c)
            gcp[c] = fire_gather(c)
        for c in range(head, n_chunk):
            clip_chunk(c)
        cp_new.wait()

        waited = set()
        for c in range(n_chunk):
            for cp in gcp[c]:
                cp.wait()
            patch(c, rows[c % NBUF])
            ocp[c] = fire_out(c)
            nxt = c + DEPTH
            if nxt < n_chunk:
                if nxt >= NBUF:
                    ocp[nxt - NBUF].wait()
                    waited.add(nxt - NBUF)
                gcp[nxt] = fire_gather(nxt)
        for j in range(n_chunk):
            if j not in waited:
                ocp[j].wait()

    return k(ids_flat, original_weight, new_weight)


def kernel(input_ids, original_weight, new_weight):
    b, s = input_ids.shape
    n_orig, d = original_weight.shape
    n_new = new_weight.shape[0]
    ids_flat = input_ids.reshape(-1).astype(jnp.int32)
    out = _lookup(ids_flat, original_weight, new_weight, n_orig, n_new)
    return out.reshape(b, s, d)


# speedup vs baseline: 1.0350x; 1.0144x over previous
import functools

import jax
import jax.numpy as jnp
from jax import lax
from jax.experimental import pallas as pl
from jax.experimental.pallas import tpu as pltpu
from jax.experimental.pallas import tpu_sc as plsc

L = 16
NBUF = 6
DEPTH = 4


@functools.partial(jax.jit, static_argnums=(3, 4))
def _lookup(ids_flat, original_weight, new_weight, n_orig, n_new):
    total = ids_flat.shape[0]
    d = original_weight.shape[1]

    info = plsc.get_sparse_core_info()
    nw = info.num_cores * info.num_subcores
    per_w = total // nw
    sub_rows = 128
    chunk = sub_rows
    n_chunk = per_w // chunk
    g_per_chunk = chunk // L

    mesh = plsc.VectorSubcoreMesh(core_axis_name="c", subcore_axis_name="s")

    @functools.partial(
        pl.kernel,
        mesh=mesh,
        compiler_params=pltpu.CompilerParams(needs_layout_passes=False),
        out_type=jax.ShapeDtypeStruct((total, d), jnp.float32),
        scratch_types=[
            pltpu.VMEM((n_new, d), jnp.float32),
            pltpu.VMEM((per_w,), jnp.int32),
            pltpu.VMEM((per_w,), jnp.int32),
            pltpu.SMEM((n_chunk,), jnp.int32),
            *([pltpu.VMEM((chunk, d), jnp.float32)] * NBUF),
            pltpu.SemaphoreType.DMA,
            pltpu.SemaphoreType.DMA,
            pltpu.SemaphoreType.DMA,
        ],
    )
    def k(ids_hbm, orig_hbm, new_hbm, out_hbm,
          new_v, raw_v, clip_v, flags_s, *rows_and_sems):
        rows = rows_and_sems[:NBUF]
        gsem, osem, nsem = rows_and_sems[NBUF:]
        wid = lax.axis_index("s") * info.num_cores + lax.axis_index("c")
        base = wid * per_w

        cp_new = pltpu.async_copy(new_hbm, new_v, nsem)
        pltpu.sync_copy(ids_hbm.at[pl.ds(base, per_w)], raw_v)

        lim = jnp.full((L,), n_orig - 1, jnp.int32)
        big = jnp.full((L,), n_orig, jnp.int32)
        zero = jnp.zeros((L,), jnp.int32)
        lane = jnp.arange(L, dtype=jnp.int32)
        one = jnp.full((L,), 1, jnp.int32)

        def clip_chunk(c):
            def clip_body(g, acc, c=c):
                off = c * chunk + g * L
                v = raw_v[pl.ds(off, L)]
                clip_v[pl.ds(off, L)] = jnp.minimum(v, lim)
                return acc + jnp.where(v >= big, one, zero)

            acc = lax.fori_loop(0, g_per_chunk, clip_body, zero)
            flags_s[c] = plsc.all_reduce_population_count(acc > 0)[0]

        def fire_gather(c):
            return [
                pltpu.async_copy(
                    orig_hbm.at[clip_v.at[pl.ds(c * chunk, chunk)]],
                    rows[c % NBUF], gsem),
            ]

        def fire_out(c):
            return pltpu.async_copy(
                rows[c % NBUF],
                out_hbm.at[pl.ds(base + c * chunk, chunk)], osem)

        def patch(c, buf):
            @pl.when(flags_s[c] > 0)
            def _():
                def group_body(g, carry):
                    v = raw_v[pl.ds(c * chunk + g * L, L)]
                    mask = v >= big

                    @pl.when(plsc.all_reduce_population_count(mask)[0] > 0)
                    def _():
                        nidx = jnp.minimum(
                            jnp.maximum(v - big, zero),
                            jnp.full((L,), n_new - 1, jnp.int32))
                        row_ids = jnp.full((L,), g * L, jnp.int32) + lane

                        def col_body(cb, cc):
                            cvec = jnp.full((L,), cb, jnp.int32)
                            vals = plsc.load_gather(new_v, [nidx, cvec],
                                                    mask=mask)
                            plsc.store_scatter(buf, [row_ids, cvec], vals,
                                               mask=mask)
                            return cc

                        lax.fori_loop(0, d, col_body, 0)

                    return carry

                lax.fori_loop(0, g_per_chunk, group_body, 0)

        gcp = [None] * n_chunk
        ocp = [None] * n_chunk
        head = min(DEPTH, n_chunk)
        for c in range(head):
            clip_chunk(c)
            gcp[c] = fire_gather(c)
        for c in range(head, n_chunk):
            clip_chunk(c)
        cp_new.wait()

        waited = set()
        for c in range(n_chunk):
            for cp in gcp[c]:
                cp.wait()
            patch(c, rows[c % NBUF])
            ocp[c] = fire_out(c)
            nxt = c + DEPTH
            if nxt < n_chunk:
                if nxt >= NBUF:
                    ocp[nxt - NBUF].wait()
                    waited.add(nxt - NBUF)
                gcp[nxt] = fire_gather(nxt)
        for j in range(n_chunk):
            if j not in waited:
                ocp[j].wait()

    return k(ids_flat, original_weight, new_weight)


def kernel(input_ids, original_weight, new_weight):
    b, s = input_ids.shape
    n_orig, d = original_weight.shape
    n_new = new_weight.shape[0]
    ids_flat = input_ids.reshape(-1).astype(jnp.int32)
    out = _lookup(ids_flat, original_weight, new_weight, n_orig, n_new)
    return out.reshape(b, s, d)
